# Initial kernel scaffold; baseline (speedup 1.0000x reference)
import jax
import jax.numpy as jnp
from jax.experimental import pallas as pl


def kernel(x, W):
    raise NotImplementedError("write your pallas kernel here")


# speedup vs baseline: 9.4011x; 9.4011x over previous
import jax
import jax.numpy as jnp
from jax import lax
from jax.experimental import pallas as pl


def _vq_body(xt_ref, wt_ref, q_ref, idx_ref):
    xb = xt_ref[0]
    wt = wt_ref[...]
    T = xb.shape[0]
    K = wt.shape[1]
    prod = lax.dot_general(xb, wt, (((1,), (0,)), ((), ())),
                           preferred_element_type=jnp.float32,
                           precision=lax.Precision.HIGHEST)
    wsq = jnp.sum(wt * wt, axis=0, keepdims=True)
    s = wsq - 2.0 * prod
    smin = jnp.min(s, axis=1, keepdims=True)
    kiota = lax.broadcasted_iota(jnp.int32, s.shape, 1)
    cand = jnp.where(s == smin, kiota, jnp.int32(K))
    idx = jnp.min(cand, axis=1)
    idx_ref[0, 0, :] = idx
    onehot_t = (lax.broadcasted_iota(jnp.int32, (K, T), 0)
                == idx[None, :]).astype(jnp.float32)
    q_ref[0] = lax.dot_general(wt, onehot_t, (((1,), (0,)), ((), ())),
                               preferred_element_type=jnp.float32,
                               precision=lax.Precision.HIGHEST)


def kernel(x, W):
    B, C, T = x.shape
    K = W.shape[0]
    xt = jnp.transpose(x, (0, 2, 1))
    wt = W.T
    q, idx3 = pl.pallas_call(
        _vq_body,
        grid=(B,),
        in_specs=[
            pl.BlockSpec((1, T, C), lambda b: (b, 0, 0)),
            pl.BlockSpec((C, K), lambda b: (0, 0)),
        ],
        out_specs=[
            pl.BlockSpec((1, C, T), lambda b: (b, 0, 0)),
            pl.BlockSpec((1, 1, T), lambda b: (b, 0, 0)),
        ],
        out_shape=[
            jax.ShapeDtypeStruct((B, C, T), jnp.float32),
            jax.ShapeDtypeStruct((B, 1, T), jnp.int32),
        ],
    )(xt, wt)
    return q, idx3.reshape(B, T)


# speedup vs baseline: 12.8400x; 1.3658x over previous
import jax
import jax.numpy as jnp
from jax import lax
from jax.experimental import pallas as pl


def _vq_body(x_ref, w_ref, q_ref, idx_ref):
    xb = x_ref[0]
    w = w_ref[...]
    K = w.shape[0]
    prod = lax.dot_general(w, xb, (((1,), (0,)), ((), ())),
                           preferred_element_type=jnp.float32,
                           precision=lax.Precision.HIGHEST)
    wsq = jnp.sum(w * w, axis=1, keepdims=True)
    s = wsq - 2.0 * prod
    smin = jnp.min(s, axis=0, keepdims=True)
    kiota = lax.broadcasted_iota(jnp.int32, s.shape, 0)
    cand = jnp.where(s == smin, kiota, jnp.int32(K))
    idx = jnp.min(cand, axis=0)
    idx_ref[0, 0, :] = idx
    onehot_t = (kiota == idx[None, :]).astype(jnp.float32)
    q_ref[0] = lax.dot_general(w, onehot_t, (((0,), (0,)), ((), ())),
                               preferred_element_type=jnp.float32,
                               precision=lax.Precision.HIGHEST)


def kernel(x, W):
    B, C, T = x.shape
    K = W.shape[0]
    q, idx3 = pl.pallas_call(
        _vq_body,
        grid=(B,),
        in_specs=[
            pl.BlockSpec((1, C, T), lambda b: (b, 0, 0)),
            pl.BlockSpec((K, C), lambda b: (0, 0)),
        ],
        out_specs=[
            pl.BlockSpec((1, C, T), lambda b: (b, 0, 0)),
            pl.BlockSpec((1, 1, T), lambda b: (b, 0, 0)),
        ],
        out_shape=[
            jax.ShapeDtypeStruct((B, C, T), jnp.float32),
            jax.ShapeDtypeStruct((B, 1, T), jnp.int32),
        ],
    )(x, W)
    return q, idx3.reshape(B, T)
